# Initial kernel scaffold; baseline (speedup 1.0000x reference)
import jax
import jax.numpy as jnp
from jax.experimental import pallas as pl


def kernel(xyz, xyz_goal, params):
    raise NotImplementedError("write your pallas kernel here")


# speedup vs baseline: 7.4864x; 7.4864x over previous
import functools

import jax
import jax.numpy as jnp
from jax.experimental import pallas as pl

EPS = 1e-5
_BN_SCALE = 1.0 / (1.0 + EPS) ** 0.5
_HI = jax.lax.Precision.HIGHEST


def _bf(v):
    return v.astype(jnp.bfloat16).astype(jnp.float32)


def _bnconv(x, p, i):
    y = jnp.dot(x, p[i][:, :], preferred_element_type=jnp.float32)
    return (y + p[i + 1][:, :]) * p[i + 2][:, :] + p[i + 3][:, :]


def _densitynet(dsc, p):
    t = _bf(dsc) * _bf(p[4][:, :])
    t = jnp.maximum((t + p[5][:, :]) * p[6][:, :] + p[7][:, :], 0.0)
    t = jnp.maximum(_bnconv(t, p, 8), 0.0)
    return jax.nn.sigmoid(_bnconv(t, p, 12))


def _weightnet(gx, p):
    wv = jnp.maximum(_bnconv(gx, p, 16), 0.0)
    wv = jnp.maximum(_bnconv(wv, p, 20), 0.0)
    return jnp.maximum(_bnconv(wv, p, 24), 0.0)


def _density_body(c1, scale, n_tiles, xt_ref, xT_ref, out_ref):
    xTv = xT_ref[0]
    norms = (xTv[0:1] * xTv[0:1] + xTv[1:2] * xTv[1:2]
             + xTv[2:3] * xTv[2:3])
    n = xTv.shape[1]
    t = n // n_tiles
    for r in range(n_tiles):
        a = xt_ref[0, r * t:(r + 1) * t, :]
        rn = (a[:, 0:1] * a[:, 0:1] + a[:, 1:2] * a[:, 1:2]
              + a[:, 2:3] * a[:, 2:3])
        cross = jnp.dot(a, xTv, preferred_element_type=jnp.float32)
        d = (-2.0 * cross + rn) + norms
        g = jnp.exp(d * (-c1))
        s = jnp.sum(g, axis=1, keepdims=True)
        out_ref[0, r * t:(r + 1) * t, :] = scale / s


def _inv_density(xt, xT, bandwidth, n_tiles):
    b, n, _ = xt.shape
    c1 = 1.0 / (2.0 * bandwidth * bandwidth)
    scale = 2.5 * bandwidth * n
    return pl.pallas_call(
        functools.partial(_density_body, c1, scale, n_tiles),
        grid=(b,),
        in_specs=[
            pl.BlockSpec((1, n, 3), lambda i: (i, 0, 0)),
            pl.BlockSpec((1, 3, n), lambda i: (i, 0, 0)),
        ],
        out_specs=pl.BlockSpec((1, n, 1), lambda i: (i, 0, 0)),
        out_shape=jax.ShapeDtypeStruct((b, n, 1), jnp.float32),
    )(xt, xT)


def _fps_body(npoint, xT_ref, nxyz_ref):
    b, _, n = xT_ref.shape
    x0 = xT_ref[:, 0, :]
    x1 = xT_ref[:, 1, :]
    x2 = xT_ref[:, 2, :]
    iota = jax.lax.broadcasted_iota(jnp.int32, (b, n), 1).astype(jnp.float32)

    def body(i, carry):
        dist, far = carry
        mask = (iota == far).astype(jnp.float32)
        c0 = jnp.sum(x0 * mask, axis=1, keepdims=True)
        c1 = jnp.sum(x1 * mask, axis=1, keepdims=True)
        c2 = jnp.sum(x2 * mask, axis=1, keepdims=True)
        nxyz_ref[pl.ds(i, 1)] = jnp.concatenate([c0, c1, c2], axis=1)[None]
        d = (x0 - c0) ** 2 + (x1 - c1) ** 2 + (x2 - c2) ** 2
        dist = jnp.minimum(dist, d)
        m = jnp.max(dist, axis=1, keepdims=True)
        far = jnp.min(jnp.where(dist == m, iota, float(n)), axis=1,
                      keepdims=True)
        return dist, far

    dist0 = jnp.full((b, n), 1e10, jnp.float32)
    far0 = jnp.zeros((b, 1), jnp.float32)
    jax.lax.fori_loop(0, npoint, body, (dist0, far0))


def _fps(xT, npoint):
    return pl.pallas_call(
        functools.partial(_fps_body, npoint),
        out_shape=jax.ShapeDtypeStruct((npoint, xT.shape[0], 3), jnp.float32),
    )(xT)


def _stage_body(nsample, cin, cmid, *refs):
    out_ref = refs[-1]
    nx_ref, xT_ref, ft_ref = refs[0], refs[1], refs[2]
    p = refs[3:-1]
    nx = nx_ref[0]
    xTv = xT_ref[0]
    ftv = ft_ref[0]
    s, _ = nx.shape
    n = xTv.shape[1]
    f = ftv.shape[1]

    norms = (xTv[0:1] * xTv[0:1] + xTv[1:2] * xTv[1:2]
             + xTv[2:3] * xTv[2:3])
    rn = (nx[:, 0:1] * nx[:, 0:1] + nx[:, 1:2] * nx[:, 1:2]
          + nx[:, 2:3] * nx[:, 2:3])
    cross = jnp.dot(nx, xTv, preferred_element_type=jnp.float32)
    dist = (-2.0 * cross + rn) + norms
    iota = jax.lax.broadcasted_iota(jnp.int32, (s, n), 1).astype(jnp.float32)

    gs = []
    for _k in range(nsample):
        m = jnp.min(dist, axis=1, keepdims=True)
        idx = jnp.min(jnp.where(dist == m, iota, float(n)), axis=1,
                      keepdims=True)
        ohb = iota == idx
        dist = jnp.where(ohb, 1e30, dist)
        gs.append(jnp.dot(ohb.astype(jnp.float32), ftv,
                          preferred_element_type=jnp.float32,
                          precision=_HI))

    dmax = gs[0][:, f - 1:f]
    for k in range(1, nsample):
        dmax = jnp.maximum(dmax, gs[k][:, f - 1:f])

    mw = [jnp.zeros((s, cmid), jnp.float32) for _ in range(16)]
    for k in range(nsample):
        g = gs[k]
        gx = g[:, 0:3] - nx
        fin = jnp.concatenate([gx, g[:, 3:f - 1]], axis=1)
        h = jnp.maximum(_bnconv(fin, p, 0), 0.0)
        d3 = _densitynet(g[:, f - 1:f] / dmax, p)
        npb = _bf(h * d3)
        wvb = _bf(_weightnet(gx, p))
        for w in range(16):
            mw[w] = mw[w] + npb * wvb[:, w:w + 1]

    acc = jnp.zeros((s, cmid), jnp.float32)
    for w in range(16):
        acc = acc + jnp.dot(mw[w], p[28][w],
                            preferred_element_type=jnp.float32)
    out_ref[0] = jnp.maximum(
        (acc + p[29][:, :]) * p[30][:, :] + p[31][:, :], 0.0)


def _stage(nx, xT, ft, wp, nsample, cin, cmid):
    b, s, _ = nx.shape
    n = xT.shape[2]
    f = ft.shape[2]
    full = lambda shp: pl.BlockSpec(shp, lambda i: tuple(0 for _ in shp))
    return pl.pallas_call(
        functools.partial(_stage_body, nsample, cin, cmid),
        grid=(b,),
        in_specs=[
            pl.BlockSpec((1, s, 3), lambda i: (i, 0, 0)),
            pl.BlockSpec((1, 3, n), lambda i: (i, 0, 0)),
            pl.BlockSpec((1, n, f), lambda i: (i, 0, 0)),
        ] + [full(w.shape) for w in wp],
        out_specs=pl.BlockSpec((1, s, cmid), lambda i: (i, 0, 0)),
        out_shape=jax.ShapeDtypeStruct((b, s, cmid), jnp.float32),
    )(nx, xT, ft, *wp)


def _ga_body(cmid, *refs):
    out_ref = refs[-1]
    p_ref, invd_ref = refs[0], refs[1]
    p = refs[2:-1]
    pv = p_ref[0]
    invd = invd_ref[0]
    h = jnp.maximum(_bnconv(pv, p, 0), 0.0)
    dmax = jnp.max(invd, axis=0, keepdims=True)
    d3 = _densitynet(invd / dmax, p)
    npb = _bf(h * d3)
    wvb = _bf(_weightnet(pv[:, 0:3], p))
    acc = jnp.zeros((1, cmid), jnp.float32)
    for w in range(16):
        mwv = jnp.sum(npb * wvb[:, w:w + 1], axis=0, keepdims=True)
        acc = acc + jnp.dot(mwv, p[28][w], preferred_element_type=jnp.float32)
    out_ref[0] = jnp.maximum(
        (acc + p[29][:, :]) * p[30][:, :] + p[31][:, :], 0.0)


def _group_all(pts, invd, wp, cmid):
    b, n, cin = pts.shape
    full = lambda shp: pl.BlockSpec(shp, lambda i: tuple(0 for _ in shp))
    return pl.pallas_call(
        functools.partial(_ga_body, cmid),
        grid=(b,),
        in_specs=[
            pl.BlockSpec((1, n, cin), lambda i: (i, 0, 0)),
            pl.BlockSpec((1, n, 1), lambda i: (i, 0, 0)),
        ] + [full(w.shape) for w in wp],
        out_specs=pl.BlockSpec((1, 1, cmid), lambda i: (i, 0, 0)),
        out_shape=jax.ShapeDtypeStruct((b, 1, cmid), jnp.float32),
    )(pts, invd, *wp)


def _head_body(enc_ref, f1w_ref, f1b_ref, g1g_ref, g1b_ref, f3w_ref, f3b_ref,
               g3g_ref, g3b_ref, f4w_ref, f4b_ref, g4g_ref, g4b_ref,
               f5w_ref, f5b_ref, out_ref):
    enc = enc_ref[:, :]
    x = enc[8:16] - enc[0:8]

    def block(x, wr, br, gr, betar):
        y = jnp.dot(x, wr[:, :], preferred_element_type=jnp.float32) + br[:, :]
        m = jnp.mean(y, axis=1, keepdims=True)
        v = jnp.mean((y - m) ** 2, axis=1, keepdims=True)
        y = (y - m) / jnp.sqrt(v + EPS) * gr[:, :] + betar[:, :]
        return jnp.maximum(y, 0.0)

    x = block(x, f1w_ref, f1b_ref, g1g_ref, g1b_ref)
    x = block(x, f3w_ref, f3b_ref, g3g_ref, g3b_ref)
    x = block(x, f4w_ref, f4b_ref, g4g_ref, g4b_ref)
    out_ref[:, :] = (jnp.dot(x, f5w_ref[:, :],
                             preferred_element_type=jnp.float32) + f5b_ref[:, :])


def _row(v):
    return v.reshape(1, -1)


def _layer(p, name):
    return [p[name + '_w'].T, _row(p[name + '_b']),
            _row(p[name + '_g'] * _BN_SCALE), _row(p[name + '_beta'])]


def _stage_params(p, cmid):
    out = (_layer(p, 'mlp0') + _layer(p, 'dn0') + _layer(p, 'dn1')
           + _layer(p, 'dn2') + _layer(p, 'wn0') + _layer(p, 'wn1')
           + _layer(p, 'wn2'))
    lw = p['lin_w'].reshape(cmid, cmid, 16).transpose(2, 1, 0)
    out += [lw, _row(p['lin_b']), _row(p['bnlin_g'] * _BN_SCALE),
            _row(p['bnlin_beta'])]
    return out


def kernel(xyz, xyz_goal, params):
    pc = jnp.concatenate([xyz, xyz_goal], axis=0)
    xT1 = pc[:, :3, :]
    pts1 = jnp.transpose(pc, (0, 2, 1))
    xt1 = pts1[:, :, 0:3]

    sa1 = _stage_params(params['sa1'], 64)
    sa2 = _stage_params(params['sa2'], 128)
    sa3 = _stage_params(params['sa3'], 256)

    invd1 = _inv_density(xt1, xT1, 0.1, 8)
    nx1 = jnp.transpose(_fps(xT1, 128), (1, 0, 2))
    ft1 = jnp.concatenate([xt1, pts1, invd1], axis=2)
    l1 = _stage(nx1, xT1, ft1, sa1, 8, 9, 64)

    xT2 = jnp.transpose(nx1, (0, 2, 1))
    invd2 = _inv_density(nx1, xT2, 0.2, 1)
    nx2 = jnp.transpose(_fps(xT2, 64), (1, 0, 2))
    ft2 = jnp.concatenate([nx1, l1, invd2], axis=2)
    l2 = _stage(nx2, xT2, ft2, sa2, 16, 67, 128)

    xT3 = jnp.transpose(nx2, (0, 2, 1))
    invd3 = _inv_density(nx2, xT3, 0.4, 1)
    p3 = jnp.concatenate([nx2, l2], axis=2)
    enc = _group_all(p3, invd3, sa3, 256)[:, 0, :]

    h = params['head']
    hw = [h['fc1_w'].T, _row(h['fc1_b']), _row(h['gn1_g']), _row(h['gn1_b']),
          h['fc3_w'].T, _row(h['fc3_b']), _row(h['gn3_g']), _row(h['gn3_b']),
          h['fc4_w'].T, _row(h['fc4_b']), _row(h['gn4_g']), _row(h['gn4_b']),
          h['fc5_w'].T, _row(h['fc5_b'])]
    out = pl.pallas_call(
        _head_body,
        out_shape=jax.ShapeDtypeStruct((8, 3), jnp.float32),
    )(enc, *hw)
    return out


# speedup vs baseline: 7.4901x; 1.0005x over previous
import functools

import jax
import jax.numpy as jnp
from jax.experimental import pallas as pl
from jax.experimental.pallas import tpu as pltpu

EPS = 1e-5
_BN_SCALE = 1.0 / (1.0 + EPS) ** 0.5
_HI = jax.lax.Precision.HIGHEST


def _bf(v):
    return v.astype(jnp.bfloat16).astype(jnp.float32)


def _bnconv(x, p, i):
    y = jnp.dot(x, p[i][:, :], preferred_element_type=jnp.float32)
    return (y + p[i + 1][:, :]) * p[i + 2][:, :] + p[i + 3][:, :]


def _densitynet(dsc, p):
    t = _bf(dsc) * _bf(p[4][:, :])
    t = jnp.maximum((t + p[5][:, :]) * p[6][:, :] + p[7][:, :], 0.0)
    t = jnp.maximum(_bnconv(t, p, 8), 0.0)
    return jax.nn.sigmoid(_bnconv(t, p, 12))


def _weightnet(gx, p):
    wv = jnp.maximum(_bnconv(gx, p, 16), 0.0)
    wv = jnp.maximum(_bnconv(wv, p, 20), 0.0)
    return jnp.maximum(_bnconv(wv, p, 24), 0.0)


def _density_body(c1, scale, n_tiles, xt_ref, xT_ref, out_ref):
    xTv = xT_ref[0]
    norms = (xTv[0:1] * xTv[0:1] + xTv[1:2] * xTv[1:2]
             + xTv[2:3] * xTv[2:3])
    n = xTv.shape[1]
    t = n // n_tiles
    for r in range(n_tiles):
        a = xt_ref[0, r * t:(r + 1) * t, :]
        rn = (a[:, 0:1] * a[:, 0:1] + a[:, 1:2] * a[:, 1:2]
              + a[:, 2:3] * a[:, 2:3])
        cross = jnp.dot(a, xTv, preferred_element_type=jnp.float32)
        d = (-2.0 * cross + rn) + norms
        g = jnp.exp(d * (-c1))
        s = jnp.sum(g, axis=1, keepdims=True)
        out_ref[0, r * t:(r + 1) * t, :] = scale / s


def _inv_density(xt, xT, bandwidth, n_tiles):
    b, n, _ = xt.shape
    c1 = 1.0 / (2.0 * bandwidth * bandwidth)
    scale = 2.5 * bandwidth * n
    return pl.pallas_call(
        functools.partial(_density_body, c1, scale, n_tiles),
        grid=(b,),
        compiler_params=pltpu.CompilerParams(
            dimension_semantics=("parallel",)),
        in_specs=[
            pl.BlockSpec((1, n, 3), lambda i: (i, 0, 0)),
            pl.BlockSpec((1, 3, n), lambda i: (i, 0, 0)),
        ],
        out_specs=pl.BlockSpec((1, n, 1), lambda i: (i, 0, 0)),
        out_shape=jax.ShapeDtypeStruct((b, n, 1), jnp.float32),
    )(xt, xT)


def _fps_body(npoint, xT_ref, nxyz_ref):
    b, _, n = xT_ref.shape
    x0 = xT_ref[:, 0, :]
    x1 = xT_ref[:, 1, :]
    x2 = xT_ref[:, 2, :]
    iota = jax.lax.broadcasted_iota(jnp.int32, (b, n), 1).astype(jnp.float32)

    def body(i, carry):
        dist, far = carry
        mask = (iota == far).astype(jnp.float32)
        c0 = jnp.sum(x0 * mask, axis=1, keepdims=True)
        c1 = jnp.sum(x1 * mask, axis=1, keepdims=True)
        c2 = jnp.sum(x2 * mask, axis=1, keepdims=True)
        nxyz_ref[pl.ds(i, 1)] = jnp.concatenate([c0, c1, c2], axis=1)[None]
        d = (x0 - c0) ** 2 + (x1 - c1) ** 2 + (x2 - c2) ** 2
        dist = jnp.minimum(dist, d)
        m = jnp.max(dist, axis=1, keepdims=True)
        far = jnp.min(jnp.where(dist == m, iota, float(n)), axis=1,
                      keepdims=True)
        return dist, far

    dist0 = jnp.full((b, n), 1e10, jnp.float32)
    far0 = jnp.zeros((b, 1), jnp.float32)
    jax.lax.fori_loop(0, npoint, body, (dist0, far0))


def _fps(xT, npoint):
    return pl.pallas_call(
        functools.partial(_fps_body, npoint),
        out_shape=jax.ShapeDtypeStruct((npoint, xT.shape[0], 3), jnp.float32),
    )(xT)


def _stage_body(nsample, cin, cmid, *refs):
    out_ref = refs[-1]
    nx_ref, xT_ref, ft_ref = refs[0], refs[1], refs[2]
    p = refs[3:-1]
    nx = nx_ref[0]
    xTv = xT_ref[0]
    ftv = ft_ref[0]
    s, _ = nx.shape
    n = xTv.shape[1]
    f = ftv.shape[1]

    norms = (xTv[0:1] * xTv[0:1] + xTv[1:2] * xTv[1:2]
             + xTv[2:3] * xTv[2:3])
    rn = (nx[:, 0:1] * nx[:, 0:1] + nx[:, 1:2] * nx[:, 1:2]
          + nx[:, 2:3] * nx[:, 2:3])
    cross = jnp.dot(nx, xTv, preferred_element_type=jnp.float32)
    dist = (-2.0 * cross + rn) + norms
    iota = jax.lax.broadcasted_iota(jnp.int32, (s, n), 1).astype(jnp.float32)

    gs = []
    for _k in range(nsample):
        m = jnp.min(dist, axis=1, keepdims=True)
        idx = jnp.min(jnp.where(dist == m, iota, float(n)), axis=1,
                      keepdims=True)
        ohb = iota == idx
        dist = jnp.where(ohb, 1e30, dist)
        gs.append(jnp.dot(ohb.astype(jnp.float32), ftv,
                          preferred_element_type=jnp.float32,
                          precision=_HI))

    dmax = gs[0][:, f - 1:f]
    for k in range(1, nsample):
        dmax = jnp.maximum(dmax, gs[k][:, f - 1:f])

    mw = [jnp.zeros((s, cmid), jnp.float32) for _ in range(16)]
    for k in range(nsample):
        g = gs[k]
        gx = g[:, 0:3] - nx
        fin = jnp.concatenate([gx, g[:, 3:f - 1]], axis=1)
        h = jnp.maximum(_bnconv(fin, p, 0), 0.0)
        d3 = _densitynet(g[:, f - 1:f] / dmax, p)
        npb = _bf(h * d3)
        wvb = _bf(_weightnet(gx, p))
        for w in range(16):
            mw[w] = mw[w] + npb * wvb[:, w:w + 1]

    acc = jnp.zeros((s, cmid), jnp.float32)
    for w in range(16):
        acc = acc + jnp.dot(mw[w], p[28][w],
                            preferred_element_type=jnp.float32)
    out_ref[0] = jnp.maximum(
        (acc + p[29][:, :]) * p[30][:, :] + p[31][:, :], 0.0)


def _stage(nx, xT, ft, wp, nsample, cin, cmid):
    b, s, _ = nx.shape
    n = xT.shape[2]
    f = ft.shape[2]
    full = lambda shp: pl.BlockSpec(shp, lambda i: tuple(0 for _ in shp))
    return pl.pallas_call(
        functools.partial(_stage_body, nsample, cin, cmid),
        grid=(b,),
        compiler_params=pltpu.CompilerParams(
            dimension_semantics=("parallel",)),
        in_specs=[
            pl.BlockSpec((1, s, 3), lambda i: (i, 0, 0)),
            pl.BlockSpec((1, 3, n), lambda i: (i, 0, 0)),
            pl.BlockSpec((1, n, f), lambda i: (i, 0, 0)),
        ] + [full(w.shape) for w in wp],
        out_specs=pl.BlockSpec((1, s, cmid), lambda i: (i, 0, 0)),
        out_shape=jax.ShapeDtypeStruct((b, s, cmid), jnp.float32),
    )(nx, xT, ft, *wp)


def _ga_body(cmid, *refs):
    out_ref = refs[-1]
    p_ref, invd_ref = refs[0], refs[1]
    p = refs[2:-1]
    pv = p_ref[0]
    invd = invd_ref[0]
    h = jnp.maximum(_bnconv(pv, p, 0), 0.0)
    dmax = jnp.max(invd, axis=0, keepdims=True)
    d3 = _densitynet(invd / dmax, p)
    npb = _bf(h * d3)
    wvb = _bf(_weightnet(pv[:, 0:3], p))
    acc = jnp.zeros((1, cmid), jnp.float32)
    for w in range(16):
        mwv = jnp.sum(npb * wvb[:, w:w + 1], axis=0, keepdims=True)
        acc = acc + jnp.dot(mwv, p[28][w], preferred_element_type=jnp.float32)
    out_ref[0] = jnp.maximum(
        (acc + p[29][:, :]) * p[30][:, :] + p[31][:, :], 0.0)


def _group_all(pts, invd, wp, cmid):
    b, n, cin = pts.shape
    full = lambda shp: pl.BlockSpec(shp, lambda i: tuple(0 for _ in shp))
    return pl.pallas_call(
        functools.partial(_ga_body, cmid),
        grid=(b,),
        compiler_params=pltpu.CompilerParams(
            dimension_semantics=("parallel",)),
        in_specs=[
            pl.BlockSpec((1, n, cin), lambda i: (i, 0, 0)),
            pl.BlockSpec((1, n, 1), lambda i: (i, 0, 0)),
        ] + [full(w.shape) for w in wp],
        out_specs=pl.BlockSpec((1, 1, cmid), lambda i: (i, 0, 0)),
        out_shape=jax.ShapeDtypeStruct((b, 1, cmid), jnp.float32),
    )(pts, invd, *wp)


def _head_body(enc_ref, f1w_ref, f1b_ref, g1g_ref, g1b_ref, f3w_ref, f3b_ref,
               g3g_ref, g3b_ref, f4w_ref, f4b_ref, g4g_ref, g4b_ref,
               f5w_ref, f5b_ref, out_ref):
    enc = enc_ref[:, :]
    x = enc[8:16] - enc[0:8]

    def block(x, wr, br, gr, betar):
        y = jnp.dot(x, wr[:, :], preferred_element_type=jnp.float32) + br[:, :]
        m = jnp.mean(y, axis=1, keepdims=True)
        v = jnp.mean((y - m) ** 2, axis=1, keepdims=True)
        y = (y - m) / jnp.sqrt(v + EPS) * gr[:, :] + betar[:, :]
        return jnp.maximum(y, 0.0)

    x = block(x, f1w_ref, f1b_ref, g1g_ref, g1b_ref)
    x = block(x, f3w_ref, f3b_ref, g3g_ref, g3b_ref)
    x = block(x, f4w_ref, f4b_ref, g4g_ref, g4b_ref)
    out_ref[:, :] = (jnp.dot(x, f5w_ref[:, :],
                             preferred_element_type=jnp.float32) + f5b_ref[:, :])


def _row(v):
    return v.reshape(1, -1)


def _layer(p, name):
    return [p[name + '_w'].T, _row(p[name + '_b']),
            _row(p[name + '_g'] * _BN_SCALE), _row(p[name + '_beta'])]


def _stage_params(p, cmid):
    out = (_layer(p, 'mlp0') + _layer(p, 'dn0') + _layer(p, 'dn1')
           + _layer(p, 'dn2') + _layer(p, 'wn0') + _layer(p, 'wn1')
           + _layer(p, 'wn2'))
    lw = p['lin_w'].reshape(cmid, cmid, 16).transpose(2, 1, 0)
    out += [lw, _row(p['lin_b']), _row(p['bnlin_g'] * _BN_SCALE),
            _row(p['bnlin_beta'])]
    return out


def kernel(xyz, xyz_goal, params):
    pc = jnp.concatenate([xyz, xyz_goal], axis=0)
    xT1 = pc[:, :3, :]
    pts1 = jnp.transpose(pc, (0, 2, 1))
    xt1 = pts1[:, :, 0:3]

    sa1 = _stage_params(params['sa1'], 64)
    sa2 = _stage_params(params['sa2'], 128)
    sa3 = _stage_params(params['sa3'], 256)

    invd1 = _inv_density(xt1, xT1, 0.1, 8)
    nx1 = jnp.transpose(_fps(xT1, 128), (1, 0, 2))
    ft1 = jnp.concatenate([xt1, pts1, invd1], axis=2)
    l1 = _stage(nx1, xT1, ft1, sa1, 8, 9, 64)

    xT2 = jnp.transpose(nx1, (0, 2, 1))
    invd2 = _inv_density(nx1, xT2, 0.2, 1)
    nx2 = jnp.transpose(_fps(xT2, 64), (1, 0, 2))
    ft2 = jnp.concatenate([nx1, l1, invd2], axis=2)
    l2 = _stage(nx2, xT2, ft2, sa2, 16, 67, 128)

    xT3 = jnp.transpose(nx2, (0, 2, 1))
    invd3 = _inv_density(nx2, xT3, 0.4, 1)
    p3 = jnp.concatenate([nx2, l2], axis=2)
    enc = _group_all(p3, invd3, sa3, 256)[:, 0, :]

    h = params['head']
    hw = [h['fc1_w'].T, _row(h['fc1_b']), _row(h['gn1_g']), _row(h['gn1_b']),
          h['fc3_w'].T, _row(h['fc3_b']), _row(h['gn3_g']), _row(h['gn3_b']),
          h['fc4_w'].T, _row(h['fc4_b']), _row(h['gn4_g']), _row(h['gn4_b']),
          h['fc5_w'].T, _row(h['fc5_b'])]
    out = pl.pallas_call(
        _head_body,
        out_shape=jax.ShapeDtypeStruct((8, 3), jnp.float32),
    )(enc, *hw)
    return out


# speedup vs baseline: 8.0035x; 1.0685x over previous
import functools

import jax
import jax.numpy as jnp
from jax.experimental import pallas as pl
from jax.experimental.pallas import tpu as pltpu

EPS = 1e-5
_BN_SCALE = 1.0 / (1.0 + EPS) ** 0.5
_HI = jax.lax.Precision.HIGHEST


def _bf(v):
    return v.astype(jnp.bfloat16).astype(jnp.float32)


def _bnconv(x, p, i):
    y = jnp.dot(x, p[i][:, :], preferred_element_type=jnp.float32)
    return (y + p[i + 1][:, :]) * p[i + 2][:, :] + p[i + 3][:, :]


def _densitynet(dsc, p):
    t = _bf(dsc) * _bf(p[4][:, :])
    t = jnp.maximum((t + p[5][:, :]) * p[6][:, :] + p[7][:, :], 0.0)
    t = jnp.maximum(_bnconv(t, p, 8), 0.0)
    return jax.nn.sigmoid(_bnconv(t, p, 12))


def _weightnet(gx, p):
    wv = jnp.maximum(_bnconv(gx, p, 16), 0.0)
    wv = jnp.maximum(_bnconv(wv, p, 20), 0.0)
    return jnp.maximum(_bnconv(wv, p, 24), 0.0)


def _density_body(c1, scale, n_tiles, xt_ref, xT_ref, out_ref):
    xTv = xT_ref[0]
    norms = (xTv[0:1] * xTv[0:1] + xTv[1:2] * xTv[1:2]
             + xTv[2:3] * xTv[2:3])
    n = xTv.shape[1]
    t = n // n_tiles
    for r in range(n_tiles):
        a = xt_ref[0, r * t:(r + 1) * t, :]
        rn = (a[:, 0:1] * a[:, 0:1] + a[:, 1:2] * a[:, 1:2]
              + a[:, 2:3] * a[:, 2:3])
        cross = jnp.dot(a, xTv, preferred_element_type=jnp.float32)
        d = (-2.0 * cross + rn) + norms
        g = jnp.exp(d * (-c1))
        s = jnp.sum(g, axis=1, keepdims=True)
        out_ref[0, r * t:(r + 1) * t, :] = scale / s


def _inv_density(xt, xT, bandwidth, n_tiles):
    b, n, _ = xt.shape
    c1 = 1.0 / (2.0 * bandwidth * bandwidth)
    scale = 2.5 * bandwidth * n
    return pl.pallas_call(
        functools.partial(_density_body, c1, scale, n_tiles),
        grid=(b,),
        compiler_params=pltpu.CompilerParams(
            dimension_semantics=("parallel",)),
        in_specs=[
            pl.BlockSpec((1, n, 3), lambda i: (i, 0, 0)),
            pl.BlockSpec((1, 3, n), lambda i: (i, 0, 0)),
        ],
        out_specs=pl.BlockSpec((1, n, 1), lambda i: (i, 0, 0)),
        out_shape=jax.ShapeDtypeStruct((b, n, 1), jnp.float32),
    )(xt, xT)


def _fps_body(npoint, xT_ref, nxyz_ref):
    b, _, n = xT_ref.shape
    x0 = xT_ref[:, 0, :]
    x1 = xT_ref[:, 1, :]
    x2 = xT_ref[:, 2, :]
    iota = jax.lax.broadcasted_iota(jnp.int32, (b, n), 1).astype(jnp.float32)

    def body(i, carry):
        dist, far = carry
        mask = (iota == far).astype(jnp.float32)
        c0 = jnp.sum(x0 * mask, axis=1, keepdims=True)
        c1 = jnp.sum(x1 * mask, axis=1, keepdims=True)
        c2 = jnp.sum(x2 * mask, axis=1, keepdims=True)
        nxyz_ref[pl.ds(i, 1)] = jnp.concatenate([c0, c1, c2], axis=1)[None]
        d = (x0 - c0) ** 2 + (x1 - c1) ** 2 + (x2 - c2) ** 2
        dist = jnp.minimum(dist, d)
        m = jnp.max(dist, axis=1, keepdims=True)
        far = jnp.min(jnp.where(dist == m, iota, float(n)), axis=1,
                      keepdims=True)
        return dist, far

    dist0 = jnp.full((b, n), 1e10, jnp.float32)
    far0 = jnp.zeros((b, 1), jnp.float32)
    jax.lax.fori_loop(0, npoint, body, (dist0, far0))


def _fps(xT, npoint):
    return pl.pallas_call(
        functools.partial(_fps_body, npoint),
        out_shape=jax.ShapeDtypeStruct((npoint, xT.shape[0], 3), jnp.float32),
    )(xT)


def _stage_body(nsample, cin, cmid, *refs):
    out_ref = refs[-1]
    nx_ref, xT_ref, ft_ref = refs[0], refs[1], refs[2]
    p = refs[3:-1]
    nx = nx_ref[0]
    xTv = xT_ref[0]
    ftv = ft_ref[0]
    s, _ = nx.shape
    n = xTv.shape[1]
    f = ftv.shape[1]

    norms = (xTv[0:1] * xTv[0:1] + xTv[1:2] * xTv[1:2]
             + xTv[2:3] * xTv[2:3])
    rn = (nx[:, 0:1] * nx[:, 0:1] + nx[:, 1:2] * nx[:, 1:2]
          + nx[:, 2:3] * nx[:, 2:3])
    cross = jnp.dot(nx, xTv, preferred_element_type=jnp.float32)
    dist = (-2.0 * cross + rn) + norms
    iota = jax.lax.broadcasted_iota(jnp.int32, (s, n), 1).astype(jnp.float32)

    ohs = []
    for _k in range(nsample):
        m = jnp.min(dist, axis=1, keepdims=True)
        idx = jnp.min(jnp.where(dist == m, iota, float(n)), axis=1,
                      keepdims=True)
        ohb = iota == idx
        dist = jnp.where(ohb, 1e30, dist)
        ohs.append(ohb.astype(jnp.float32))

    oh = jnp.concatenate(ohs, axis=0)
    gb = jnp.dot(oh, ftv, preferred_element_type=jnp.float32,
                 precision=_HI)

    dmax = gb[0:s, f - 1:f]
    for k in range(1, nsample):
        dmax = jnp.maximum(dmax, gb[k * s:(k + 1) * s, f - 1:f])
    dmaxt = jnp.concatenate([dmax] * nsample, axis=0)
    nxt = jnp.concatenate([nx] * nsample, axis=0)

    gx = gb[:, 0:3] - nxt
    fin = jnp.concatenate([gx, gb[:, 3:f - 1]], axis=1)
    h = jnp.maximum(_bnconv(fin, p, 0), 0.0)
    d3 = _densitynet(gb[:, f - 1:f] / dmaxt, p)
    npb = _bf(h * d3)
    wvb = _bf(_weightnet(gx, p))

    acc = jnp.zeros((s, cmid), jnp.float32)
    for w in range(16):
        full = npb * wvb[:, w:w + 1]
        mww = full[0:s]
        for k in range(1, nsample):
            mww = mww + full[k * s:(k + 1) * s]
        acc = acc + jnp.dot(mww, p[28][w],
                            preferred_element_type=jnp.float32)
    out_ref[0] = jnp.maximum(
        (acc + p[29][:, :]) * p[30][:, :] + p[31][:, :], 0.0)


def _stage(nx, xT, ft, wp, nsample, cin, cmid):
    b, s, _ = nx.shape
    n = xT.shape[2]
    f = ft.shape[2]
    full = lambda shp: pl.BlockSpec(shp, lambda i: tuple(0 for _ in shp))
    return pl.pallas_call(
        functools.partial(_stage_body, nsample, cin, cmid),
        grid=(b,),
        compiler_params=pltpu.CompilerParams(
            dimension_semantics=("parallel",)),
        in_specs=[
            pl.BlockSpec((1, s, 3), lambda i: (i, 0, 0)),
            pl.BlockSpec((1, 3, n), lambda i: (i, 0, 0)),
            pl.BlockSpec((1, n, f), lambda i: (i, 0, 0)),
        ] + [full(w.shape) for w in wp],
        out_specs=pl.BlockSpec((1, s, cmid), lambda i: (i, 0, 0)),
        out_shape=jax.ShapeDtypeStruct((b, s, cmid), jnp.float32),
    )(nx, xT, ft, *wp)


def _ga_body(cmid, *refs):
    out_ref = refs[-1]
    p_ref, invd_ref = refs[0], refs[1]
    p = refs[2:-1]
    pv = p_ref[0]
    invd = invd_ref[0]
    h = jnp.maximum(_bnconv(pv, p, 0), 0.0)
    dmax = jnp.max(invd, axis=0, keepdims=True)
    d3 = _densitynet(invd / dmax, p)
    npb = _bf(h * d3)
    wvb = _bf(_weightnet(pv[:, 0:3], p))
    acc = jnp.zeros((1, cmid), jnp.float32)
    for w in range(16):
        mwv = jnp.sum(npb * wvb[:, w:w + 1], axis=0, keepdims=True)
        acc = acc + jnp.dot(mwv, p[28][w], preferred_element_type=jnp.float32)
    out_ref[0] = jnp.maximum(
        (acc + p[29][:, :]) * p[30][:, :] + p[31][:, :], 0.0)


def _group_all(pts, invd, wp, cmid):
    b, n, cin = pts.shape
    full = lambda shp: pl.BlockSpec(shp, lambda i: tuple(0 for _ in shp))
    return pl.pallas_call(
        functools.partial(_ga_body, cmid),
        grid=(b,),
        compiler_params=pltpu.CompilerParams(
            dimension_semantics=("parallel",)),
        in_specs=[
            pl.BlockSpec((1, n, cin), lambda i: (i, 0, 0)),
            pl.BlockSpec((1, n, 1), lambda i: (i, 0, 0)),
        ] + [full(w.shape) for w in wp],
        out_specs=pl.BlockSpec((1, 1, cmid), lambda i: (i, 0, 0)),
        out_shape=jax.ShapeDtypeStruct((b, 1, cmid), jnp.float32),
    )(pts, invd, *wp)


def _head_body(enc_ref, f1w_ref, f1b_ref, g1g_ref, g1b_ref, f3w_ref, f3b_ref,
               g3g_ref, g3b_ref, f4w_ref, f4b_ref, g4g_ref, g4b_ref,
               f5w_ref, f5b_ref, out_ref):
    enc = enc_ref[:, :]
    x = enc[8:16] - enc[0:8]

    def block(x, wr, br, gr, betar):
        y = jnp.dot(x, wr[:, :], preferred_element_type=jnp.float32) + br[:, :]
        m = jnp.mean(y, axis=1, keepdims=True)
        v = jnp.mean((y - m) ** 2, axis=1, keepdims=True)
        y = (y - m) / jnp.sqrt(v + EPS) * gr[:, :] + betar[:, :]
        return jnp.maximum(y, 0.0)

    x = block(x, f1w_ref, f1b_ref, g1g_ref, g1b_ref)
    x = block(x, f3w_ref, f3b_ref, g3g_ref, g3b_ref)
    x = block(x, f4w_ref, f4b_ref, g4g_ref, g4b_ref)
    out_ref[:, :] = (jnp.dot(x, f5w_ref[:, :],
                             preferred_element_type=jnp.float32) + f5b_ref[:, :])


def _row(v):
    return v.reshape(1, -1)


def _layer(p, name):
    return [p[name + '_w'].T, _row(p[name + '_b']),
            _row(p[name + '_g'] * _BN_SCALE), _row(p[name + '_beta'])]


def _stage_params(p, cmid):
    out = (_layer(p, 'mlp0') + _layer(p, 'dn0') + _layer(p, 'dn1')
           + _layer(p, 'dn2') + _layer(p, 'wn0') + _layer(p, 'wn1')
           + _layer(p, 'wn2'))
    lw = p['lin_w'].reshape(cmid, cmid, 16).transpose(2, 1, 0)
    out += [lw, _row(p['lin_b']), _row(p['bnlin_g'] * _BN_SCALE),
            _row(p['bnlin_beta'])]
    return out


def kernel(xyz, xyz_goal, params):
    pc = jnp.concatenate([xyz, xyz_goal], axis=0)
    xT1 = pc[:, :3, :]
    pts1 = jnp.transpose(pc, (0, 2, 1))
    xt1 = pts1[:, :, 0:3]

    sa1 = _stage_params(params['sa1'], 64)
    sa2 = _stage_params(params['sa2'], 128)
    sa3 = _stage_params(params['sa3'], 256)

    invd1 = _inv_density(xt1, xT1, 0.1, 8)
    nx1 = jnp.transpose(_fps(xT1, 128), (1, 0, 2))
    ft1 = jnp.concatenate([xt1, pts1, invd1], axis=2)
    l1 = _stage(nx1, xT1, ft1, sa1, 8, 9, 64)

    xT2 = jnp.transpose(nx1, (0, 2, 1))
    invd2 = _inv_density(nx1, xT2, 0.2, 1)
    nx2 = jnp.transpose(_fps(xT2, 64), (1, 0, 2))
    ft2 = jnp.concatenate([nx1, l1, invd2], axis=2)
    l2 = _stage(nx2, xT2, ft2, sa2, 16, 67, 128)

    xT3 = jnp.transpose(nx2, (0, 2, 1))
    invd3 = _inv_density(nx2, xT3, 0.4, 1)
    p3 = jnp.concatenate([nx2, l2], axis=2)
    enc = _group_all(p3, invd3, sa3, 256)[:, 0, :]

    h = params['head']
    hw = [h['fc1_w'].T, _row(h['fc1_b']), _row(h['gn1_g']), _row(h['gn1_b']),
          h['fc3_w'].T, _row(h['fc3_b']), _row(h['gn3_g']), _row(h['gn3_b']),
          h['fc4_w'].T, _row(h['fc4_b']), _row(h['gn4_g']), _row(h['gn4_b']),
          h['fc5_w'].T, _row(h['fc5_b'])]
    out = pl.pallas_call(
        _head_body,
        out_shape=jax.ShapeDtypeStruct((8, 3), jnp.float32),
    )(enc, *hw)
    return out


# speedup vs baseline: 8.2283x; 1.0281x over previous
import functools

import jax
import jax.numpy as jnp
from jax.experimental import pallas as pl
from jax.experimental.pallas import tpu as pltpu

EPS = 1e-5
_BN_SCALE = 1.0 / (1.0 + EPS) ** 0.5
_HI = jax.lax.Precision.HIGHEST


def _bf(v):
    return v.astype(jnp.bfloat16).astype(jnp.float32)


def _bnconv(x, p, i):
    y = jnp.dot(x, p[i][:, :], preferred_element_type=jnp.float32)
    return (y + p[i + 1][:, :]) * p[i + 2][:, :] + p[i + 3][:, :]


def _densitynet(dsc, p):
    t = _bf(dsc) * _bf(p[4][:, :])
    t = jnp.maximum((t + p[5][:, :]) * p[6][:, :] + p[7][:, :], 0.0)
    t = jnp.maximum(_bnconv(t, p, 8), 0.0)
    return jax.nn.sigmoid(_bnconv(t, p, 12))


def _weightnet(gx, p):
    wv = jnp.maximum(_bnconv(gx, p, 16), 0.0)
    wv = jnp.maximum(_bnconv(wv, p, 20), 0.0)
    return jnp.maximum(_bnconv(wv, p, 24), 0.0)


def _density_body(c1, scale, n_tiles, xt_ref, xT_ref, out_ref):
    xTv = xT_ref[0]
    norms = (xTv[0:1] * xTv[0:1] + xTv[1:2] * xTv[1:2]
             + xTv[2:3] * xTv[2:3])
    n = xTv.shape[1]
    t = n // n_tiles
    for r in range(n_tiles):
        a = xt_ref[0, r * t:(r + 1) * t, :]
        rn = (a[:, 0:1] * a[:, 0:1] + a[:, 1:2] * a[:, 1:2]
              + a[:, 2:3] * a[:, 2:3])
        cross = jnp.dot(a, xTv, preferred_element_type=jnp.float32)
        d = (-2.0 * cross + rn) + norms
        g = jnp.exp(d * (-c1))
        s = jnp.sum(g, axis=1, keepdims=True)
        out_ref[0, r * t:(r + 1) * t, :] = scale / s


def _inv_density(xt, xT, bandwidth, n_tiles):
    b, n, _ = xt.shape
    c1 = 1.0 / (2.0 * bandwidth * bandwidth)
    scale = 2.5 * bandwidth * n
    return pl.pallas_call(
        functools.partial(_density_body, c1, scale, n_tiles),
        grid=(b,),
        compiler_params=pltpu.CompilerParams(
            dimension_semantics=("parallel",)),
        in_specs=[
            pl.BlockSpec((1, n, 3), lambda i: (i, 0, 0)),
            pl.BlockSpec((1, 3, n), lambda i: (i, 0, 0)),
        ],
        out_specs=pl.BlockSpec((1, n, 1), lambda i: (i, 0, 0)),
        out_shape=jax.ShapeDtypeStruct((b, n, 1), jnp.float32),
    )(xt, xT)


def _fps_body(npoint, xT_ref, nxyz_ref):
    b, _, n = xT_ref.shape
    x0 = xT_ref[:, 0, :]
    x1 = xT_ref[:, 1, :]
    x2 = xT_ref[:, 2, :]
    iota = jax.lax.broadcasted_iota(jnp.int32, (b, n), 1).astype(jnp.float32)

    def body(i, carry):
        dist, far = carry
        mask = (iota == far).astype(jnp.float32)
        c0 = jnp.sum(x0 * mask, axis=1, keepdims=True)
        c1 = jnp.sum(x1 * mask, axis=1, keepdims=True)
        c2 = jnp.sum(x2 * mask, axis=1, keepdims=True)
        nxyz_ref[pl.ds(i, 1)] = jnp.concatenate([c0, c1, c2], axis=1)[None]
        d = (x0 - c0) ** 2 + (x1 - c1) ** 2 + (x2 - c2) ** 2
        dist = jnp.minimum(dist, d)
        m = jnp.max(dist, axis=1, keepdims=True)
        far = jnp.min(jnp.where(dist == m, iota, float(n)), axis=1,
                      keepdims=True)
        return dist, far

    dist0 = jnp.full((b, n), 1e10, jnp.float32)
    far0 = jnp.zeros((b, 1), jnp.float32)
    jax.lax.fori_loop(0, npoint, body, (dist0, far0))


def _fps(xT, npoint):
    return pl.pallas_call(
        functools.partial(_fps_body, npoint),
        out_shape=jax.ShapeDtypeStruct((npoint, xT.shape[0], 3), jnp.float32),
    )(xT)


def _stage_body(nsample, cin, cmid, *refs):
    out_ref = refs[-1]
    nx_ref, xT_ref, ft_ref = refs[0], refs[1], refs[2]
    p = refs[3:-1]
    nx = nx_ref[0]
    xTv = xT_ref[0]
    ftv = ft_ref[0]
    s, _ = nx.shape
    n = xTv.shape[1]
    f = ftv.shape[1]

    norms = (xTv[0:1] * xTv[0:1] + xTv[1:2] * xTv[1:2]
             + xTv[2:3] * xTv[2:3])
    rn = (nx[:, 0:1] * nx[:, 0:1] + nx[:, 1:2] * nx[:, 1:2]
          + nx[:, 2:3] * nx[:, 2:3])
    cross = jnp.dot(nx, xTv, preferred_element_type=jnp.float32)
    dist = (-2.0 * cross + rn) + norms
    iota = jax.lax.broadcasted_iota(jnp.int32, (s, n), 1).astype(jnp.float32)

    ohs = []
    for _k in range(nsample):
        m = jnp.min(dist, axis=1, keepdims=True)
        idx = jnp.min(jnp.where(dist == m, iota, float(n)), axis=1,
                      keepdims=True)
        ohb = iota == idx
        dist = jnp.where(ohb, 1e30, dist)
        ohs.append(ohb.astype(jnp.float32))

    gb = jnp.concatenate(
        [jnp.dot(o, ftv, preferred_element_type=jnp.float32, precision=_HI)
         for o in ohs], axis=0)

    dmax = gb[0:s, f - 1:f]
    for k in range(1, nsample):
        dmax = jnp.maximum(dmax, gb[k * s:(k + 1) * s, f - 1:f])
    dmaxt = jnp.concatenate([dmax] * nsample, axis=0)
    nxt = jnp.concatenate([nx] * nsample, axis=0)

    gx = gb[:, 0:3] - nxt
    fin = jnp.concatenate([gx, gb[:, 3:f - 1]], axis=1)
    h = jnp.maximum(_bnconv(fin, p, 0), 0.0)
    d3 = _densitynet(gb[:, f - 1:f] / dmaxt, p)
    npb = _bf(h * d3)
    wvb = _bf(_weightnet(gx, p))

    acc = jnp.zeros((s, cmid), jnp.float32)
    for w in range(16):
        full = npb * wvb[:, w:w + 1]
        mww = full[0:s]
        for k in range(1, nsample):
            mww = mww + full[k * s:(k + 1) * s]
        acc = acc + jnp.dot(mww, p[28][w],
                            preferred_element_type=jnp.float32)
    out_ref[0] = jnp.maximum(
        (acc + p[29][:, :]) * p[30][:, :] + p[31][:, :], 0.0)


def _stage(nx, xT, ft, wp, nsample, cin, cmid):
    b, s, _ = nx.shape
    n = xT.shape[2]
    f = ft.shape[2]
    full = lambda shp: pl.BlockSpec(shp, lambda i: tuple(0 for _ in shp))
    return pl.pallas_call(
        functools.partial(_stage_body, nsample, cin, cmid),
        grid=(b,),
        compiler_params=pltpu.CompilerParams(
            dimension_semantics=("parallel",)),
        in_specs=[
            pl.BlockSpec((1, s, 3), lambda i: (i, 0, 0)),
            pl.BlockSpec((1, 3, n), lambda i: (i, 0, 0)),
            pl.BlockSpec((1, n, f), lambda i: (i, 0, 0)),
        ] + [full(w.shape) for w in wp],
        out_specs=pl.BlockSpec((1, s, cmid), lambda i: (i, 0, 0)),
        out_shape=jax.ShapeDtypeStruct((b, s, cmid), jnp.float32),
    )(nx, xT, ft, *wp)


def _ga_body(cmid, *refs):
    out_ref = refs[-1]
    p_ref, invd_ref = refs[0], refs[1]
    p = refs[2:-1]
    pv = p_ref[0]
    invd = invd_ref[0]
    h = jnp.maximum(_bnconv(pv, p, 0), 0.0)
    dmax = jnp.max(invd, axis=0, keepdims=True)
    d3 = _densitynet(invd / dmax, p)
    npb = _bf(h * d3)
    wvb = _bf(_weightnet(pv[:, 0:3], p))
    acc = jnp.zeros((1, cmid), jnp.float32)
    for w in range(16):
        mwv = jnp.sum(npb * wvb[:, w:w + 1], axis=0, keepdims=True)
        acc = acc + jnp.dot(mwv, p[28][w], preferred_element_type=jnp.float32)
    out_ref[0] = jnp.maximum(
        (acc + p[29][:, :]) * p[30][:, :] + p[31][:, :], 0.0)


def _group_all(pts, invd, wp, cmid):
    b, n, cin = pts.shape
    full = lambda shp: pl.BlockSpec(shp, lambda i: tuple(0 for _ in shp))
    return pl.pallas_call(
        functools.partial(_ga_body, cmid),
        grid=(b,),
        compiler_params=pltpu.CompilerParams(
            dimension_semantics=("parallel",)),
        in_specs=[
            pl.BlockSpec((1, n, cin), lambda i: (i, 0, 0)),
            pl.BlockSpec((1, n, 1), lambda i: (i, 0, 0)),
        ] + [full(w.shape) for w in wp],
        out_specs=pl.BlockSpec((1, 1, cmid), lambda i: (i, 0, 0)),
        out_shape=jax.ShapeDtypeStruct((b, 1, cmid), jnp.float32),
    )(pts, invd, *wp)


def _head_body(enc_ref, f1w_ref, f1b_ref, g1g_ref, g1b_ref, f3w_ref, f3b_ref,
               g3g_ref, g3b_ref, f4w_ref, f4b_ref, g4g_ref, g4b_ref,
               f5w_ref, f5b_ref, out_ref):
    enc = enc_ref[:, :]
    x = enc[8:16] - enc[0:8]

    def block(x, wr, br, gr, betar):
        y = jnp.dot(x, wr[:, :], preferred_element_type=jnp.float32) + br[:, :]
        m = jnp.mean(y, axis=1, keepdims=True)
        v = jnp.mean((y - m) ** 2, axis=1, keepdims=True)
        y = (y - m) / jnp.sqrt(v + EPS) * gr[:, :] + betar[:, :]
        return jnp.maximum(y, 0.0)

    x = block(x, f1w_ref, f1b_ref, g1g_ref, g1b_ref)
    x = block(x, f3w_ref, f3b_ref, g3g_ref, g3b_ref)
    x = block(x, f4w_ref, f4b_ref, g4g_ref, g4b_ref)
    out_ref[:, :] = (jnp.dot(x, f5w_ref[:, :],
                             preferred_element_type=jnp.float32) + f5b_ref[:, :])


def _row(v):
    return v.reshape(1, -1)


def _layer(p, name):
    return [p[name + '_w'].T, _row(p[name + '_b']),
            _row(p[name + '_g'] * _BN_SCALE), _row(p[name + '_beta'])]


def _stage_params(p, cmid):
    out = (_layer(p, 'mlp0') + _layer(p, 'dn0') + _layer(p, 'dn1')
           + _layer(p, 'dn2') + _layer(p, 'wn0') + _layer(p, 'wn1')
           + _layer(p, 'wn2'))
    lw = p['lin_w'].reshape(cmid, cmid, 16).transpose(2, 1, 0)
    out += [lw, _row(p['lin_b']), _row(p['bnlin_g'] * _BN_SCALE),
            _row(p['bnlin_beta'])]
    return out


def kernel(xyz, xyz_goal, params):
    pc = jnp.concatenate([xyz, xyz_goal], axis=0)
    xT1 = pc[:, :3, :]
    pts1 = jnp.transpose(pc, (0, 2, 1))
    xt1 = pts1[:, :, 0:3]

    sa1 = _stage_params(params['sa1'], 64)
    sa2 = _stage_params(params['sa2'], 128)
    sa3 = _stage_params(params['sa3'], 256)

    invd1 = _inv_density(xt1, xT1, 0.1, 8)
    nx1 = jnp.transpose(_fps(xT1, 128), (1, 0, 2))
    ft1 = jnp.concatenate([xt1, pts1, invd1], axis=2)
    l1 = _stage(nx1, xT1, ft1, sa1, 8, 9, 64)

    xT2 = jnp.transpose(nx1, (0, 2, 1))
    invd2 = _inv_density(nx1, xT2, 0.2, 1)
    nx2 = jnp.transpose(_fps(xT2, 64), (1, 0, 2))
    ft2 = jnp.concatenate([nx1, l1, invd2], axis=2)
    l2 = _stage(nx2, xT2, ft2, sa2, 16, 67, 128)

    xT3 = jnp.transpose(nx2, (0, 2, 1))
    invd3 = _inv_density(nx2, xT3, 0.4, 1)
    p3 = jnp.concatenate([nx2, l2], axis=2)
    enc = _group_all(p3, invd3, sa3, 256)[:, 0, :]

    h = params['head']
    hw = [h['fc1_w'].T, _row(h['fc1_b']), _row(h['gn1_g']), _row(h['gn1_b']),
          h['fc3_w'].T, _row(h['fc3_b']), _row(h['gn3_g']), _row(h['gn3_b']),
          h['fc4_w'].T, _row(h['fc4_b']), _row(h['gn4_g']), _row(h['gn4_b']),
          h['fc5_w'].T, _row(h['fc5_b'])]
    out = pl.pallas_call(
        _head_body,
        out_shape=jax.ShapeDtypeStruct((8, 3), jnp.float32),
    )(enc, *hw)
    return out
